# baseline (speedup 1.0000x reference)
import jax
import jax.numpy as jnp
from jax import lax
from jax.experimental import pallas as pl
from jax.experimental.pallas import tpu as pltpu
from jax.experimental.pallas import tpu_sc as plsc

_L = 16


def _sc_window_mean_body(emb_hbm, out_hbm, tab_v, out_v):
    S = out_v.shape[0]
    B = out_hbm.shape[0]
    w = lax.axis_index("c") * 16 + lax.axis_index("s")
    col = w * _L

    pltpu.sync_copy(emb_hbm.at[:, pl.ds(col, _L)], tab_v)

    inv = jnp.float32(1.0 / S)

    def init_body(j, s):
        return s + tab_v[j]

    s0 = lax.fori_loop(0, S, init_body, jnp.zeros((_L,), jnp.float32))
    out_v[0] = s0 * inv

    def roll_body(i, s):
        s = s + tab_v[i + (S - 1)] - tab_v[i - 1]
        out_v[i] = s * inv
        return s

    lax.fori_loop(1, S, roll_body, s0)

    for b in range(B):
        pltpu.sync_copy(out_v, out_hbm.at[b, :, pl.ds(col, _L)])


def kernel(x, embeddings):
    B, S, H = x.shape
    k = pl.kernel(
        _sc_window_mean_body,
        out_type=jax.ShapeDtypeStruct((B, S, H), jnp.float32),
        mesh=plsc.VectorSubcoreMesh(core_axis_name="c", subcore_axis_name="s"),
        scratch_types=[
            pltpu.VMEM((2 * S - 1, _L), jnp.float32),
            pltpu.VMEM((S, _L), jnp.float32),
        ],
        compiler_params=pltpu.CompilerParams(use_tc_tiling_on_sc=False),
    )
    return k(embeddings)
